# baseline (speedup 1.0000x reference)
import functools

import jax
import jax.numpy as jnp
from jax import lax
from jax.experimental import pallas as pl
from jax.experimental.pallas import tpu as pltpu
from jax.experimental.pallas import tpu_sc as plsc

N = 10000
E = 320000
H = 128
R = 64
F = 128
CUTOFF = 5.0
LOG2 = 0.6931471805599453

NC = 2
NS = 16
NW = NC * NS
EPW = E // NW
C = 80
NCHUNK = EPW // C
NP = 10240
RPT = NP // NS
ZB = 128


def _ssp(v):
    return jnp.maximum(v, 0.0) + jnp.log1p(jnp.exp(-jnp.abs(v))) - LOG2


EB = 3200

def _edge_filter_body(ea_ref, ew_ref, w1_ref, b1_ref, w2_ref, b2_ref, out_ref):
    ew = ew_ref[...]
    d = jnp.sqrt(jnp.sum(ew * ew, axis=1, keepdims=True))
    u = d * (1.0 / CUTOFF)
    env = jnp.where(u < 1.0, 1.0 - 3.0 * u * u + 2.0 * u * u * u, 0.0)
    ea = ea_ref[...].astype(jnp.bfloat16)
    w1 = w1_ref[...].astype(jnp.bfloat16)
    h1 = _ssp(jnp.dot(ea, w1, preferred_element_type=jnp.float32) + b1_ref[...])
    h1b = h1.astype(jnp.bfloat16)
    w2 = w2_ref[...].astype(jnp.bfloat16)
    out_ref[...] = (jnp.dot(h1b, w2, preferred_element_type=jnp.float32)
                    + b2_ref[...]) * env


def _edge_filter(edge_attr, edge_weight, w1, b1, w2, b2):
    grid = E // EB
    return pl.pallas_call(
        _edge_filter_body,
        grid=(grid,),
        in_specs=[
            pl.BlockSpec((EB, R), lambda g: (g, 0)),
            pl.BlockSpec((EB, 3), lambda g: (g, 0)),
            pl.BlockSpec((R, F), lambda g: (0, 0)),
            pl.BlockSpec((1, F), lambda g: (0, 0)),
            pl.BlockSpec((F, F), lambda g: (0, 0)),
            pl.BlockSpec((1, F), lambda g: (0, 0)),
        ],
        out_specs=pl.BlockSpec((EB, F), lambda g: (g, 0)),
        out_shape=jax.ShapeDtypeStruct((E, F), jnp.float32),
    )(edge_attr, edge_weight, w1, b1.reshape(1, F), w2, b2.reshape(1, F))


def _xw_body(x_ref, w_ref, o_ref):
    o_ref[...] = jnp.dot(x_ref[...], w_ref[...],
                         preferred_element_type=jnp.float32)


def _node_transform(x, lin1_w):
    return pl.pallas_call(
        _xw_body,
        out_shape=jax.ShapeDtypeStruct((N, F), jnp.float32),
    )(x, lin1_w)


def _sc_body(w_hbm, xw_hbm, jidx_hbm, iidx_hbm, zeros_hbm, out_hbm,
             jv0, jv1, iv0, iv1, wv0, wv1, rv0, rv1, agg_sh,
             sem_ij0, sem_ij1, sem_w0, sem_w1, sem_g0, sem_g1):
    cid = lax.axis_index("c")
    sid = lax.axis_index("s")
    wid = cid * NS + sid
    row0 = sid * RPT
    jv = (jv0, jv1)
    iv = (iv0, iv1)
    wv = (wv0, wv1)
    rv = (rv0, rv1)
    sem_ij = (sem_ij0, sem_ij1)
    sem_w = (sem_w0, sem_w1)
    sem_g = (sem_g0, sem_g1)
    ebase = wid * EPW

    for t in range(RPT // ZB):
        pltpu.sync_copy(zeros_hbm, agg_sh.at[pl.ds(row0 + t * ZB, ZB)])
    plsc.subcore_barrier()

    def _issue_idx(k, b):
        pltpu.async_copy(jidx_hbm.at[pl.ds(ebase + k * C, C)], jv[b], sem_ij[b])
        pltpu.async_copy(iidx_hbm.at[pl.ds(ebase + k * C, C)], iv[b], sem_ij[b])

    def _wait_idx(b):
        pltpu.make_async_copy(jidx_hbm.at[pl.ds(0, C)], jv[b], sem_ij[b]).wait()
        pltpu.make_async_copy(iidx_hbm.at[pl.ds(0, C)], iv[b], sem_ij[b]).wait()

    def _issue_wg(k, b):
        pltpu.async_copy(w_hbm.at[pl.ds(ebase + k * C, C)], wv[b], sem_w[b])
        pltpu.async_copy(xw_hbm.at[jv[b]], rv[b], sem_g[b])

    def _wait_wg(b):
        pltpu.make_async_copy(w_hbm.at[pl.ds(0, C)], wv[b], sem_w[b]).wait()
        pltpu.make_async_copy(xw_hbm.at[jv[b]], rv[b], sem_g[b]).wait()

    def _consume(b):
        def _mul(r, t2):
            for q in range(F // 16):
                s = pl.ds(q * 16, 16)
                wv[b][r, s] = wv[b][r, s] * rv[b][r, s]
            return t2
        lax.fori_loop(0, C, _mul, 0, unroll=2)
        pltpu.sync_copy(wv[b], agg_sh.at[iv[b]], add=True)

    def _stage(k, b, nb):
        _wait_idx(nb)
        _issue_wg(k + 1, nb)
        _wait_wg(b)
        _consume(b)

        @pl.when(k + 2 < NCHUNK)
        def _():
            _issue_idx(k + 2, b)

    pltpu.sync_copy(jidx_hbm.at[pl.ds(ebase, C)], jv0)
    pltpu.sync_copy(iidx_hbm.at[pl.ds(ebase, C)], iv0)
    _issue_wg(0, 0)
    _issue_idx(1, 1)

    def _pair(t, carry):
        k = 2 * t
        _stage(k, 0, 1)
        _stage(k + 1, 1, 0)
        return carry

    lax.fori_loop(0, (NCHUNK - 1) // 2, _pair, 0)
    _wait_wg(0)
    _consume(0)
    plsc.subcore_barrier()

    pltpu.sync_copy(agg_sh.at[pl.ds(row0, RPT)],
                    out_hbm.at[cid, pl.ds(row0, RPT)])


def _sc_aggregate(w_edges, xw, jidx, iidx, zeros_blk):
    mesh = plsc.VectorSubcoreMesh(core_axis_name="c", subcore_axis_name="s",
                                  num_cores=NC, num_subcores=NS)
    kern = functools.partial(
        pl.kernel,
        out_type=jax.ShapeDtypeStruct((NC, NP, F), jnp.float32),
        mesh=mesh,
        scratch_types=[
            pltpu.VMEM((C,), jnp.int32),
            pltpu.VMEM((C,), jnp.int32),
            pltpu.VMEM((C,), jnp.int32),
            pltpu.VMEM((C,), jnp.int32),
            pltpu.VMEM((C, F), jnp.float32),
            pltpu.VMEM((C, F), jnp.float32),
            pltpu.VMEM((C, F), jnp.float32),
            pltpu.VMEM((C, F), jnp.float32),
            pltpu.VMEM_SHARED((NP, F), jnp.float32),
            pltpu.SemaphoreType.DMA,
            pltpu.SemaphoreType.DMA,
            pltpu.SemaphoreType.DMA,
            pltpu.SemaphoreType.DMA,
            pltpu.SemaphoreType.DMA,
            pltpu.SemaphoreType.DMA,
        ],
    )(_sc_body)
    return kern(w_edges, xw, jidx, iidx, zeros_blk)


NB = 2000

def _final_body(x_ref, p_ref, w_ref, b_ref, o_ref):
    agg = p_ref[0] + p_ref[1]
    h = jnp.dot(agg, w_ref[...], preferred_element_type=jnp.float32) + b_ref[...]
    o_ref[...] = x_ref[...] + _ssp(h)


def _final(x, partials, lin2_w, lin2_b):
    grid = N // NB
    return pl.pallas_call(
        _final_body,
        grid=(grid,),
        in_specs=[
            pl.BlockSpec((NB, H), lambda g: (g, 0)),
            pl.BlockSpec((NC, NB, F), lambda g: (0, g, 0)),
            pl.BlockSpec((F, H), lambda g: (0, 0)),
            pl.BlockSpec((1, H), lambda g: (0, 0)),
        ],
        out_specs=pl.BlockSpec((NB, H), lambda g: (g, 0)),
        out_shape=jax.ShapeDtypeStruct((N, H), jnp.float32),
    )(x, partials, lin2_w, lin2_b.reshape(1, H))


def kernel(x, edge_index, edge_weight, edge_attr,
           mlp_w1, mlp_b1, mlp_w2, mlp_b2, lin1_w, lin2_w, lin2_b):
    w_edges = _edge_filter(edge_attr, edge_weight, mlp_w1, mlp_b1, mlp_w2, mlp_b2)
    xw = _node_transform(x, lin1_w)
    iidx = edge_index[0]
    jidx = edge_index[1]
    zeros_blk = jnp.zeros((ZB, F), jnp.float32)
    partials = _sc_aggregate(w_edges, xw, jidx, iidx, zeros_blk)
    return _final(x, partials, lin2_w, lin2_b)


# speedup vs baseline: 1.0232x; 1.0232x over previous
import functools

import jax
import jax.numpy as jnp
from jax import lax
from jax.experimental import pallas as pl
from jax.experimental.pallas import tpu as pltpu
from jax.experimental.pallas import tpu_sc as plsc

N = 10000
E = 320000
H = 128
R = 64
F = 128
CUTOFF = 5.0
LOG2 = 0.6931471805599453

NC = 2
NS = 16
NW = NC * NS
EPW = E // NW
C = 80
NCHUNK = EPW // C
NP = 10240
RPT = NP // NS
ZB = 128


LOG2E = 1.4426950408889634

def _ssp(v):
    y = jnp.exp2(jnp.abs(v) * (-LOG2E))
    return jnp.maximum(v, 0.0) + jnp.log2(0.5 + 0.5 * y) * LOG2


EB = 3200

def _edge_filter_body(ea_ref, ew_ref, w1_ref, b1_ref, w2_ref, b2_ref, out_ref):
    ew = ew_ref[...]
    ones3 = jnp.ones((3, 1), jnp.float32)
    s = jnp.dot(ew * ew, ones3,
                preferred_element_type=jnp.float32) * (1.0 / (CUTOFF * CUTOFF))
    s = jnp.maximum(s, 1e-30)
    u = s * lax.rsqrt(s)
    t = 1.0 - u
    env = jnp.where(u < 1.0, t * t * (1.0 + 2.0 * u), 0.0)
    ea = ea_ref[...].astype(jnp.bfloat16)
    w1 = w1_ref[...].astype(jnp.bfloat16)
    h1 = _ssp(jnp.dot(ea, w1, preferred_element_type=jnp.float32) + b1_ref[...])
    h1b = h1.astype(jnp.bfloat16)
    w2 = w2_ref[...].astype(jnp.bfloat16)
    out_ref[...] = (jnp.dot(h1b, w2, preferred_element_type=jnp.float32)
                    + b2_ref[...]) * env


def _edge_filter(edge_attr, edge_weight, w1, b1, w2, b2):
    grid = E // EB
    return pl.pallas_call(
        _edge_filter_body,
        grid=(grid,),
        in_specs=[
            pl.BlockSpec((EB, R), lambda g: (g, 0)),
            pl.BlockSpec((EB, 3), lambda g: (g, 0)),
            pl.BlockSpec((R, F), lambda g: (0, 0)),
            pl.BlockSpec((1, F), lambda g: (0, 0)),
            pl.BlockSpec((F, F), lambda g: (0, 0)),
            pl.BlockSpec((1, F), lambda g: (0, 0)),
        ],
        out_specs=pl.BlockSpec((EB, F), lambda g: (g, 0)),
        out_shape=jax.ShapeDtypeStruct((E, F), jnp.float32),
    )(edge_attr, edge_weight, w1, b1.reshape(1, F), w2, b2.reshape(1, F))


def _xw_body(x_ref, w_ref, o_ref):
    o_ref[...] = jnp.dot(x_ref[...], w_ref[...],
                         preferred_element_type=jnp.float32)


def _node_transform(x, lin1_w):
    return pl.pallas_call(
        _xw_body,
        out_shape=jax.ShapeDtypeStruct((N, F), jnp.float32),
    )(x, lin1_w)


def _sc_body(w_hbm, xw_hbm, jidx_hbm, iidx_hbm, zeros_hbm, out_hbm,
             jv0, jv1, iv0, iv1, wv0, wv1, rv0, rv1, agg_sh,
             sem_ij0, sem_ij1, sem_w0, sem_w1, sem_g0, sem_g1):
    cid = lax.axis_index("c")
    sid = lax.axis_index("s")
    wid = cid * NS + sid
    row0 = sid * RPT
    jv = (jv0, jv1)
    iv = (iv0, iv1)
    wv = (wv0, wv1)
    rv = (rv0, rv1)
    sem_ij = (sem_ij0, sem_ij1)
    sem_w = (sem_w0, sem_w1)
    sem_g = (sem_g0, sem_g1)
    ebase = wid * EPW

    for t in range(RPT // ZB):
        pltpu.sync_copy(zeros_hbm, agg_sh.at[pl.ds(row0 + t * ZB, ZB)])
    plsc.subcore_barrier()

    def _issue_idx(k, b):
        pltpu.async_copy(jidx_hbm.at[pl.ds(ebase + k * C, C)], jv[b], sem_ij[b])
        pltpu.async_copy(iidx_hbm.at[pl.ds(ebase + k * C, C)], iv[b], sem_ij[b])

    def _wait_idx(b):
        pltpu.make_async_copy(jidx_hbm.at[pl.ds(0, C)], jv[b], sem_ij[b]).wait()
        pltpu.make_async_copy(iidx_hbm.at[pl.ds(0, C)], iv[b], sem_ij[b]).wait()

    def _issue_wg(k, b):
        pltpu.async_copy(w_hbm.at[pl.ds(ebase + k * C, C)], wv[b], sem_w[b])
        pltpu.async_copy(xw_hbm.at[jv[b]], rv[b], sem_g[b])

    def _wait_wg(b):
        pltpu.make_async_copy(w_hbm.at[pl.ds(0, C)], wv[b], sem_w[b]).wait()
        pltpu.make_async_copy(w_hbm.at[pl.ds(0, C)], rv[b], sem_g[b]).wait()

    def _consume(b):
        def _mul(r, t2):
            for q in range(F // 16):
                s = pl.ds(q * 16, 16)
                wv[b][r, s] = wv[b][r, s] * rv[b][r, s]
            return t2
        lax.fori_loop(0, C, _mul, 0, unroll=2)
        pltpu.sync_copy(wv[b], agg_sh.at[iv[b]], add=True)

    def _stage(k, b, nb):
        _wait_idx(nb)
        _issue_wg(k + 1, nb)
        _wait_wg(b)
        _consume(b)

        @pl.when(k + 2 < NCHUNK)
        def _():
            _issue_idx(k + 2, b)

    pltpu.sync_copy(jidx_hbm.at[pl.ds(ebase, C)], jv0)
    pltpu.sync_copy(iidx_hbm.at[pl.ds(ebase, C)], iv0)
    _issue_wg(0, 0)
    _issue_idx(1, 1)

    def _pair(t, carry):
        k = 2 * t
        _stage(k, 0, 1)
        _stage(k + 1, 1, 0)
        return carry

    lax.fori_loop(0, (NCHUNK - 1) // 2, _pair, 0)
    _wait_wg(0)
    _consume(0)
    plsc.subcore_barrier()

    pltpu.sync_copy(agg_sh.at[pl.ds(row0, RPT)],
                    out_hbm.at[cid, pl.ds(row0, RPT)])


def _sc_aggregate(w_edges, xw, jidx, iidx, zeros_blk):
    mesh = plsc.VectorSubcoreMesh(core_axis_name="c", subcore_axis_name="s",
                                  num_cores=NC, num_subcores=NS)
    kern = functools.partial(
        pl.kernel,
        out_type=jax.ShapeDtypeStruct((NC, NP, F), jnp.float32),
        mesh=mesh,
        scratch_types=[
            pltpu.VMEM((C,), jnp.int32),
            pltpu.VMEM((C,), jnp.int32),
            pltpu.VMEM((C,), jnp.int32),
            pltpu.VMEM((C,), jnp.int32),
            pltpu.VMEM((C, F), jnp.float32),
            pltpu.VMEM((C, F), jnp.float32),
            pltpu.VMEM((C, F), jnp.float32),
            pltpu.VMEM((C, F), jnp.float32),
            pltpu.VMEM_SHARED((NP, F), jnp.float32),
            pltpu.SemaphoreType.DMA,
            pltpu.SemaphoreType.DMA,
            pltpu.SemaphoreType.DMA,
            pltpu.SemaphoreType.DMA,
            pltpu.SemaphoreType.DMA,
            pltpu.SemaphoreType.DMA,
        ],
    )(_sc_body)
    return kern(w_edges, xw, jidx, iidx, zeros_blk)


NB = 2000

def _final_body(x_ref, p_ref, w_ref, b_ref, o_ref):
    agg = p_ref[0] + p_ref[1]
    h = jnp.dot(agg, w_ref[...], preferred_element_type=jnp.float32) + b_ref[...]
    o_ref[...] = x_ref[...] + _ssp(h)


def _final(x, partials, lin2_w, lin2_b):
    grid = N // NB
    return pl.pallas_call(
        _final_body,
        grid=(grid,),
        in_specs=[
            pl.BlockSpec((NB, H), lambda g: (g, 0)),
            pl.BlockSpec((NC, NB, F), lambda g: (0, g, 0)),
            pl.BlockSpec((F, H), lambda g: (0, 0)),
            pl.BlockSpec((1, H), lambda g: (0, 0)),
        ],
        out_specs=pl.BlockSpec((NB, H), lambda g: (g, 0)),
        out_shape=jax.ShapeDtypeStruct((N, H), jnp.float32),
    )(x, partials, lin2_w, lin2_b.reshape(1, H))


def kernel(x, edge_index, edge_weight, edge_attr,
           mlp_w1, mlp_b1, mlp_w2, mlp_b2, lin1_w, lin2_w, lin2_b):
    w_edges = _edge_filter(edge_attr, edge_weight, mlp_w1, mlp_b1, mlp_w2, mlp_b2)
    xw = _node_transform(x, lin1_w)
    iidx = edge_index[0]
    jidx = edge_index[1]
    zeros_blk = jnp.zeros((ZB, F), jnp.float32)
    partials = _sc_aggregate(w_edges, xw, jidx, iidx, zeros_blk)
    return _final(x, partials, lin2_w, lin2_b)
